# Initial kernel scaffold; baseline (speedup 1.0000x reference)
import functools

import jax
import jax.numpy as jnp
from jax import lax
from jax.experimental import pallas as pl
from jax.experimental.pallas import tpu as pltpu
from jax.experimental.pallas import tpu_sc as plsc

N_NODES = 10000
N_EDGES = 320000
D = 128
N_GRAPHS = 64

N_PAD = 10240
NC, NS = 2, 16
NW = NC * NS
CHUNK = 128
NCHUNK = 79
E_PER_W = CHUNK * NCHUNK
E_PAD = E_PER_W * NW
ROWS_PER_TILE = N_PAD // NS

_mesh = plsc.VectorSubcoreMesh(core_axis_name="c", subcore_axis_name="s")


@functools.partial(
    pl.kernel,
    out_type=jax.ShapeDtypeStruct((NC, N_PAD, 16), jnp.float32),
    mesh=_mesh,
    scratch_types=[
        pltpu.VMEM((CHUNK,), jnp.int32),
        pltpu.VMEM((CHUNK, 16), jnp.float32),
        pltpu.VMEM_SHARED((N_PAD, 16), jnp.float32),
    ],
)
def _sc_degree(dst_hbm, ones_hbm, zeros_hbm, cnt_hbm, dst_v, ones_v, acc):
    c = lax.axis_index("c")
    s = lax.axis_index("s")
    wid = c * NS + s
    base = wid * E_PER_W
    r0 = s * ROWS_PER_TILE
    pltpu.sync_copy(zeros_hbm.at[pl.ds(r0, ROWS_PER_TILE)],
                    acc.at[pl.ds(r0, ROWS_PER_TILE)])
    pltpu.sync_copy(ones_hbm, ones_v)
    plsc.subcore_barrier()

    def body(i, carry):
        off = base + i * CHUNK
        pltpu.sync_copy(dst_hbm.at[pl.ds(off, CHUNK)], dst_v)
        pltpu.sync_copy(ones_v, acc.at[dst_v], add=True)
        return carry

    lax.fori_loop(0, NCHUNK, body, 0)
    plsc.subcore_barrier()
    pltpu.sync_copy(acc.at[pl.ds(r0, ROWS_PER_TILE)],
                    cnt_hbm.at[c, pl.ds(r0, ROWS_PER_TILE)])


@functools.partial(
    pl.kernel,
    out_type=jax.ShapeDtypeStruct((NC, N_PAD, D), jnp.float32),
    mesh=_mesh,
    scratch_types=[
        pltpu.VMEM((CHUNK,), jnp.int32),
        pltpu.VMEM((CHUNK,), jnp.int32),
        pltpu.VMEM((CHUNK, D), jnp.float32),
        pltpu.VMEM_SHARED((N_PAD, D), jnp.float32),
        pltpu.SemaphoreType.DMA,
    ],
)
def _sc_gather_scatter(hn_hbm, src_hbm, dst_hbm, zeros_hbm, out_hbm,
                       src_v, dst_v, rows_v, acc, sem):
    c = lax.axis_index("c")
    s = lax.axis_index("s")
    wid = c * NS + s
    base = wid * E_PER_W
    r0 = s * ROWS_PER_TILE
    pltpu.sync_copy(zeros_hbm.at[pl.ds(r0, ROWS_PER_TILE)],
                    acc.at[pl.ds(r0, ROWS_PER_TILE)])
    plsc.subcore_barrier()

    def body(i, carry):
        off = base + i * CHUNK
        pltpu.sync_copy(src_hbm.at[pl.ds(off, CHUNK)], src_v)
        pltpu.sync_copy(dst_hbm.at[pl.ds(off, CHUNK)], dst_v)
        pltpu.async_copy(hn_hbm.at[src_v], rows_v, sem).wait()
        pltpu.sync_copy(rows_v, acc.at[dst_v], add=True)
        return carry

    lax.fori_loop(0, NCHUNK, body, 0)
    plsc.subcore_barrier()
    pltpu.sync_copy(acc.at[pl.ds(r0, ROWS_PER_TILE)],
                    out_hbm.at[c, pl.ds(r0, ROWS_PER_TILE)])


def _tc1_body(cnt_ref, x_ref, w_ref, hn_ref, dis_ref):
    cnt = cnt_ref[0, :, 0:1] + cnt_ref[1, :, 0:1]
    deg = cnt + 1.0
    rid = lax.broadcasted_iota(jnp.int32, (N_PAD, 1), 0)
    dis = jnp.where(rid < N_NODES, lax.rsqrt(deg), 0.0)
    dis_ref[...] = dis
    t = jnp.dot(x_ref[...], w_ref[...], preferred_element_type=jnp.float32)
    hn_ref[...] = t * dis


def _tc_mid_body(s_ref, hn_ref, dis_ref, b_ref, w_ref, hn2_ref):
    dis = dis_ref[...]
    y = (s_ref[0] + s_ref[1] + hn_ref[...]) * dis + b_ref[...]
    h = jnp.maximum(y, 0.0)
    t = jnp.dot(h, w_ref[...], preferred_element_type=jnp.float32)
    hn2_ref[...] = t * dis


def _tc_final_body(s_ref, hn_ref, dis_ref, b_ref, batch_ref,
                   wl1t_ref, bl1_ref, wl2t_ref, bl2_ref, out_ref):
    dis = dis_ref[...]
    y = (s_ref[0] + s_ref[1] + hn_ref[...]) * dis + b_ref[...]
    h = jnp.maximum(y, 0.0)
    gid = lax.broadcasted_iota(jnp.int32, (N_PAD, N_GRAPHS), 1)
    m = (batch_ref[...] == gid).astype(jnp.float32)
    gt = lax.dot_general(h, m, (((0,), (0,)), ((), ())),
                         preferred_element_type=jnp.float32)
    counts = jnp.sum(m, axis=0, keepdims=True)
    gt = gt / jnp.maximum(counts, 1.0)
    zt = jnp.maximum(
        jnp.dot(wl1t_ref[...], gt, preferred_element_type=jnp.float32)
        + bl1_ref[...], 0.0)
    out_ref[...] = (
        jnp.dot(wl2t_ref[...], zt, preferred_element_type=jnp.float32)
        + bl2_ref[...])


_tc1 = pl.pallas_call(
    _tc1_body,
    out_shape=(jax.ShapeDtypeStruct((N_PAD, D), jnp.float32),
               jax.ShapeDtypeStruct((N_PAD, 1), jnp.float32)),
)

_tc_mid = pl.pallas_call(
    _tc_mid_body,
    out_shape=jax.ShapeDtypeStruct((N_PAD, D), jnp.float32),
)

_tc_final = pl.pallas_call(
    _tc_final_body,
    out_shape=jax.ShapeDtypeStruct((D, N_GRAPHS), jnp.float32),
)


def kernel(x, edge_index, batch, W1, b1, W2, b2, W3, b3, Wl1, bl1, Wl2, bl2):
    f32 = jnp.float32
    src = edge_index[0].astype(jnp.int32)
    dst = edge_index[1].astype(jnp.int32)
    pad = jnp.full((E_PAD - N_EDGES,), N_NODES, dtype=jnp.int32)
    src_p = jnp.concatenate([src, pad])
    dst_p = jnp.concatenate([dst, pad])

    x_p = jnp.zeros((N_PAD, D), f32).at[:N_NODES].set(x.astype(f32))
    batch_col = jnp.full((N_PAD, 1), N_GRAPHS, jnp.int32) \
        .at[:N_NODES, 0].set(batch.astype(jnp.int32))

    zeros128 = jnp.zeros((N_PAD, D), f32)
    zeros16 = jnp.zeros((N_PAD, 16), f32)
    ones16 = jnp.ones((CHUNK, 16), f32)

    b1r = b1.reshape(1, D)
    b2r = b2.reshape(1, D)
    b3r = b3.reshape(1, D)
    wl1t = Wl1.T
    bl1c = bl1.reshape(N_GRAPHS, 1)
    wl2t = jnp.zeros((D, N_GRAPHS), f32).at[:Wl2.shape[1]].set(Wl2.T)
    bl2c = jnp.zeros((D, 1), f32).at[:Wl2.shape[1], 0].set(bl2)

    cnt = _sc_degree(dst_p, ones16, zeros16)
    hn1, dis = _tc1(cnt, x_p, W1)
    s1 = _sc_gather_scatter(hn1, src_p, dst_p, zeros128)
    hn2 = _tc_mid(s1, hn1, dis, b1r, W2)
    s2 = _sc_gather_scatter(hn2, src_p, dst_p, zeros128)
    hn3 = _tc_mid(s2, hn2, dis, b2r, W3)
    s3 = _sc_gather_scatter(hn3, src_p, dst_p, zeros128)
    out_t = _tc_final(s3, hn3, dis, b3r, batch_col, wl1t, bl1c, wl2t, bl2c)
    return out_t[:Wl2.shape[1], :].T


# speedup vs baseline: 8.5561x; 8.5561x over previous
import functools

import jax
import jax.numpy as jnp
from jax import lax
from jax.experimental import pallas as pl
from jax.experimental.pallas import tpu as pltpu
from jax.experimental.pallas import tpu_sc as plsc

N_NODES = 10000
N_EDGES = 320000
D = 128
N_GRAPHS = 64

N_PAD = 10112
NC, NS = 2, 16
NW = NC * NS
CHUNK = 128
F_CHUNK = 128
S_CHUNK = 32
TOT_CHUNK = 2656
E_PAD = TOT_CHUNK * CHUNK
DEG_NCHUNK = 79
E_DEG_PAD = DEG_NCHUNK * NW * CHUNK
ROWS_PER_TILE = N_PAD // NS

_mesh = plsc.VectorSubcoreMesh(core_axis_name="c", subcore_axis_name="s")


@functools.partial(
    pl.kernel,
    out_type=jax.ShapeDtypeStruct((NC, N_PAD, D), jnp.float32),
    mesh=_mesh,
    scratch_types=[
        pltpu.VMEM((DEG_NCHUNK, CHUNK), jnp.int32),
        pltpu.VMEM((CHUNK, D), jnp.float32),
        pltpu.VMEM_SHARED((N_PAD, D), jnp.float32),
    ],
)
def _sc_degree(dst_hbm, ones_hbm, zeros_hbm, cnt_hbm, dst_v, ones_v, acc):
    c = lax.axis_index("c")
    s = lax.axis_index("s")
    wid = c * NS + s
    r0 = s * ROWS_PER_TILE
    pltpu.sync_copy(zeros_hbm.at[pl.ds(r0, ROWS_PER_TILE)],
                    acc.at[pl.ds(r0, ROWS_PER_TILE)])
    pltpu.sync_copy(ones_hbm, ones_v)
    pltpu.sync_copy(dst_hbm.at[wid], dst_v)
    plsc.subcore_barrier()

    def body(i, carry):
        pltpu.sync_copy(ones_v, acc.at[dst_v.at[i]], add=True)
        return carry

    lax.fori_loop(0, DEG_NCHUNK, body, 0)
    plsc.subcore_barrier()
    pltpu.sync_copy(acc.at[pl.ds(r0, ROWS_PER_TILE)],
                    cnt_hbm.at[c, pl.ds(r0, ROWS_PER_TILE)])


@functools.partial(
    pl.kernel,
    out_type=jax.ShapeDtypeStruct((NC, N_PAD, D), jnp.float32),
    mesh=_mesh,
    scratch_types=[
        pltpu.VMEM((F_CHUNK, CHUNK), jnp.int32),
        pltpu.VMEM((CHUNK,), jnp.int32),
        pltpu.VMEM((CHUNK,), jnp.int32),
        pltpu.VMEM((CHUNK, D), jnp.float32),
        pltpu.VMEM((CHUNK, D), jnp.float32),
        pltpu.VMEM_SHARED((N_PAD, D), jnp.float32),
        pltpu.SemaphoreType.DMA,
        pltpu.SemaphoreType.DMA,
        pltpu.SemaphoreType.DMA,
        pltpu.SemaphoreType.DMA,
        pltpu.SemaphoreType.DMA,
    ],
)
def _sc_gather_scatter(hn_hbm, src_hbm, dst_hbm, zeros_hbm, out_hbm,
                       dst_v, src0_v, src1_v, rows0_v, rows1_v, acc,
                       gsem0, gsem1, isem0, isem1, ssem):
    c = lax.axis_index("c")
    s = lax.axis_index("s")
    r0 = s * ROWS_PER_TILE
    base = jnp.where(c == 0, s * F_CHUNK, NS * F_CHUNK + s * S_CHUNK)
    nchunk = jnp.where(c == 0, F_CHUNK, S_CHUNK)
    pltpu.sync_copy(zeros_hbm.at[pl.ds(r0, ROWS_PER_TILE)],
                    acc.at[pl.ds(r0, ROWS_PER_TILE)])
    pltpu.sync_copy(dst_hbm.at[pl.ds(base, F_CHUNK)], dst_v)
    plsc.subcore_barrier()

    rows = (rows0_v, rows1_v)
    srcs = (src0_v, src1_v)
    gsems = (gsem0, gsem1)
    isems = (isem0, isem1)

    hn_c = hn_hbm.at[c]
    pltpu.sync_copy(src_hbm.at[pl.ds(base * CHUNK, CHUNK)], src0_v)
    pltpu.sync_copy(src_hbm.at[pl.ds((base + 1) * CHUNK, CHUNK)], src1_v)
    pltpu.async_copy(hn_c.at[src0_v], rows[0], gsems[0])
    pltpu.async_copy(hn_c.at[src1_v], rows[1], gsems[1])

    def body(i2, carry):
        for j in range(2):
            i = i2 + j
            b = rows[j]
            sv = srcs[j]
            pltpu.make_async_copy(hn_c.at[sv], b, gsems[j]).wait()

            @pl.when(i + 2 < nchunk)
            def _():
                pltpu.async_copy(
                    src_hbm.at[pl.ds((base + i + 2) * CHUNK, CHUNK)],
                    sv, isems[j])

            sc = pltpu.async_copy(b, acc.at[dst_v.at[i]], ssem, add=True)
            sc.wait()

            @pl.when(i + 2 < nchunk)
            def _():
                pltpu.make_async_copy(
                    src_hbm.at[pl.ds((base + i + 2) * CHUNK, CHUNK)],
                    sv, isems[j]).wait()
                pltpu.async_copy(hn_c.at[sv], b, gsems[j])
        return carry

    lax.fori_loop(0, nchunk // 2, lambda k, cc: body(k * 2, cc), 0)
    plsc.subcore_barrier()
    pltpu.sync_copy(acc.at[pl.ds(r0, ROWS_PER_TILE)],
                    out_hbm.at[c, pl.ds(r0, ROWS_PER_TILE)])


def _tc1_body(cnt_ref, x_ref, w_ref, hn_ref, dis_ref):
    cnt = cnt_ref[0, :, 0:1] + cnt_ref[1, :, 0:1]
    deg = cnt + 1.0
    rid = lax.broadcasted_iota(jnp.int32, (N_PAD, 1), 0)
    dis = jnp.where(rid < N_NODES, lax.rsqrt(deg), 0.0)
    dis_ref[...] = dis
    t = jnp.dot(x_ref[...], w_ref[...], preferred_element_type=jnp.float32)
    hn = t * dis
    hn_ref[0] = hn
    hn_ref[1] = hn


def _tc_mid_body(s_ref, hn_ref, dis_ref, b_ref, w_ref, hn2_ref):
    dis = dis_ref[...]
    y = (s_ref[0] + s_ref[1] + hn_ref[0]) * dis + b_ref[...]
    h = jnp.maximum(y, 0.0)
    t = jnp.dot(h, w_ref[...], preferred_element_type=jnp.float32)
    hn2 = t * dis
    hn2_ref[0] = hn2
    hn2_ref[1] = hn2


def _tc_final_body(s_ref, hn_ref, dis_ref, b_ref, batch_ref,
                   wl1t_ref, bl1_ref, wl2t_ref, bl2_ref, out_ref):
    dis = dis_ref[...]
    y = (s_ref[0] + s_ref[1] + hn_ref[0]) * dis + b_ref[...]
    h = jnp.maximum(y, 0.0)
    gid = lax.broadcasted_iota(jnp.int32, (N_PAD, N_GRAPHS), 1)
    m = (batch_ref[...] == gid).astype(jnp.float32)
    gt = lax.dot_general(h, m, (((0,), (0,)), ((), ())),
                         preferred_element_type=jnp.float32)
    counts = jnp.sum(m, axis=0, keepdims=True)
    gt = gt / jnp.maximum(counts, 1.0)
    zt = jnp.maximum(
        jnp.dot(wl1t_ref[...], gt, preferred_element_type=jnp.float32)
        + bl1_ref[...], 0.0)
    out_ref[...] = (
        jnp.dot(wl2t_ref[...], zt, preferred_element_type=jnp.float32)
        + bl2_ref[...])


_tc1 = pl.pallas_call(
    _tc1_body,
    out_shape=(jax.ShapeDtypeStruct((2, N_PAD, D), jnp.float32),
               jax.ShapeDtypeStruct((N_PAD, 1), jnp.float32)),
)

_tc_mid = pl.pallas_call(
    _tc_mid_body,
    out_shape=jax.ShapeDtypeStruct((2, N_PAD, D), jnp.float32),
)

_tc_final = pl.pallas_call(
    _tc_final_body,
    out_shape=jax.ShapeDtypeStruct((D, N_GRAPHS), jnp.float32),
)


def kernel(x, edge_index, batch, W1, b1, W2, b2, W3, b3, Wl1, bl1, Wl2, bl2):
    f32 = jnp.float32
    src = edge_index[0].astype(jnp.int32)
    dst = edge_index[1].astype(jnp.int32)
    pad = jnp.full((E_PAD - N_EDGES,), N_NODES, dtype=jnp.int32)
    src_p = jnp.concatenate([src, pad])
    dst_p = jnp.concatenate([dst, pad]).reshape(TOT_CHUNK, CHUNK)
    pad_deg = jnp.full((E_DEG_PAD - N_EDGES,), N_NODES, dtype=jnp.int32)
    dst_deg = jnp.concatenate([dst, pad_deg]).reshape(NW, DEG_NCHUNK, CHUNK)

    x_p = jnp.zeros((N_PAD, D), f32).at[:N_NODES].set(x.astype(f32))
    batch_col = jnp.full((N_PAD, 1), N_GRAPHS, jnp.int32) \
        .at[:N_NODES, 0].set(batch.astype(jnp.int32))

    zeros128 = jnp.zeros((N_PAD, D), f32)
    ones128 = jnp.ones((CHUNK, D), f32)

    b1r = b1.reshape(1, D)
    b2r = b2.reshape(1, D)
    b3r = b3.reshape(1, D)
    wl1t = Wl1.T
    bl1c = bl1.reshape(N_GRAPHS, 1)
    wl2t = jnp.zeros((D, N_GRAPHS), f32).at[:Wl2.shape[1]].set(Wl2.T)
    bl2c = jnp.zeros((D, 1), f32).at[:Wl2.shape[1], 0].set(bl2)

    cnt = _sc_degree(dst_deg, ones128, zeros128)
    hn1, dis = _tc1(cnt, x_p, W1)
    s1 = _sc_gather_scatter(hn1, src_p, dst_p, zeros128)
    hn2 = _tc_mid(s1, hn1, dis, b1r, W2)
    s2 = _sc_gather_scatter(hn2, src_p, dst_p, zeros128)
    hn3 = _tc_mid(s2, hn2, dis, b2r, W3)
    s3 = _sc_gather_scatter(hn3, src_p, dst_p, zeros128)
    out_t = _tc_final(s3, hn3, dis, b3r, batch_col, wl1t, bl1c, wl2t, bl2c)
    return out_t[:Wl2.shape[1], :].T


# speedup vs baseline: 9.4378x; 1.1031x over previous
import functools

import jax
import jax.numpy as jnp
from jax import lax
from jax.experimental import pallas as pl
from jax.experimental.pallas import tpu as pltpu
from jax.experimental.pallas import tpu_sc as plsc

N_NODES = 10000
N_EDGES = 320000
D = 128
N_GRAPHS = 64

N_PAD = 10112
NC, NS = 2, 16
NW = NC * NS
CHUNK = 128
F_CHUNK = 144
S_CHUNK = 16
TOT_CHUNK = 2704
E_PAD = TOT_CHUNK * CHUNK
DEG_NCHUNK = 79
E_DEG_PAD = DEG_NCHUNK * NW * CHUNK
ROWS_PER_TILE = N_PAD // NS

_mesh = plsc.VectorSubcoreMesh(core_axis_name="c", subcore_axis_name="s")


@functools.partial(
    pl.kernel,
    out_type=jax.ShapeDtypeStruct((NC, N_PAD, D), jnp.float32),
    mesh=_mesh,
    scratch_types=[
        pltpu.VMEM((DEG_NCHUNK, CHUNK), jnp.int32),
        pltpu.VMEM((CHUNK, D), jnp.float32),
        pltpu.VMEM_SHARED((N_PAD, D), jnp.float32),
    ],
)
def _sc_degree(dst_hbm, ones_hbm, zeros_hbm, cnt_hbm, dst_v, ones_v, acc):
    c = lax.axis_index("c")
    s = lax.axis_index("s")
    wid = c * NS + s
    r0 = s * ROWS_PER_TILE
    pltpu.sync_copy(zeros_hbm.at[pl.ds(r0, ROWS_PER_TILE)],
                    acc.at[pl.ds(r0, ROWS_PER_TILE)])
    pltpu.sync_copy(ones_hbm, ones_v)
    pltpu.sync_copy(dst_hbm.at[wid], dst_v)
    plsc.subcore_barrier()

    def body(i, carry):
        pltpu.sync_copy(ones_v, acc.at[dst_v.at[i]], add=True)
        return carry

    lax.fori_loop(0, DEG_NCHUNK, body, 0)
    plsc.subcore_barrier()
    pltpu.sync_copy(acc.at[pl.ds(r0, ROWS_PER_TILE)],
                    cnt_hbm.at[c, pl.ds(r0, ROWS_PER_TILE)])


@functools.partial(
    pl.kernel,
    out_type=jax.ShapeDtypeStruct((NC, N_PAD, D), jnp.float32),
    mesh=_mesh,
    scratch_types=[
        pltpu.VMEM((CHUNK,), jnp.int32),
        pltpu.VMEM((CHUNK,), jnp.int32),
        pltpu.VMEM((CHUNK,), jnp.int32),
        pltpu.VMEM((CHUNK,), jnp.int32),
        pltpu.VMEM((CHUNK, D), jnp.float32),
        pltpu.VMEM((CHUNK, D), jnp.float32),
        pltpu.VMEM_SHARED((N_PAD, D), jnp.float32),
        pltpu.SemaphoreType.DMA,
        pltpu.SemaphoreType.DMA,
        pltpu.SemaphoreType.DMA,
        pltpu.SemaphoreType.DMA,
        pltpu.SemaphoreType.DMA,
        pltpu.SemaphoreType.DMA,
        pltpu.SemaphoreType.DMA,
    ],
)
def _sc_gather_scatter(hn_hbm, src_hbm, dst_hbm, zeros_hbm, out_hbm,
                       src0_v, src1_v, dst0_v, dst1_v, rows0_v, rows1_v, acc,
                       gsem0, gsem1, isem0, isem1, dsem0, dsem1, ssem):
    c = lax.axis_index("c")
    s = lax.axis_index("s")
    r0 = s * ROWS_PER_TILE
    base = jnp.where(c == 0, s * F_CHUNK, NS * F_CHUNK + s * S_CHUNK)
    nchunk = jnp.where(c == 0, F_CHUNK, S_CHUNK)
    pltpu.sync_copy(zeros_hbm.at[pl.ds(r0, ROWS_PER_TILE)],
                    acc.at[pl.ds(r0, ROWS_PER_TILE)])
    plsc.subcore_barrier()

    rows = (rows0_v, rows1_v)
    srcs = (src0_v, src1_v)
    dsts = (dst0_v, dst1_v)
    gsems = (gsem0, gsem1)
    isems = (isem0, isem1)
    dsems = (dsem0, dsem1)

    def src_slice(i):
        return src_hbm.at[pl.ds((base + i) * CHUNK, CHUNK)]

    def dst_slice(i):
        return dst_hbm.at[pl.ds((base + i) * CHUNK, CHUNK)]

    hn_c = hn_hbm.at[c]
    pltpu.sync_copy(src_slice(0), src0_v)
    pltpu.sync_copy(src_slice(1), src1_v)
    pltpu.sync_copy(dst_slice(0), dst0_v)
    pltpu.sync_copy(dst_slice(1), dst1_v)
    pltpu.async_copy(hn_c.at[src0_v], rows[0], gsems[0])
    pltpu.async_copy(hn_c.at[src1_v], rows[1], gsems[1])

    def body(i2, carry):
        for j in range(2):
            i = i2 + j
            b = rows[j]
            sv = srcs[j]
            dv = dsts[j]
            pltpu.make_async_copy(hn_c.at[sv], b, gsems[j]).wait()

            @pl.when(i >= 2)
            def _():
                pltpu.make_async_copy(dst_slice(i), dv, dsems[j]).wait()

            sc = pltpu.async_copy(b, acc.at[dv], ssem, add=True)

            @pl.when(i + 2 < nchunk)
            def _():
                pltpu.async_copy(src_slice(i + 2), sv, isems[j])

            sc.wait()

            @pl.when(i + 2 < nchunk)
            def _():
                pltpu.async_copy(dst_slice(i + 2), dv, dsems[j])
                pltpu.make_async_copy(src_slice(i + 2), sv, isems[j]).wait()
                pltpu.async_copy(hn_c.at[sv], b, gsems[j])
        return carry

    lax.fori_loop(0, nchunk // 2, lambda k, cc: body(k * 2, cc), 0)
    plsc.subcore_barrier()
    pltpu.sync_copy(acc.at[pl.ds(r0, ROWS_PER_TILE)],
                    out_hbm.at[c, pl.ds(r0, ROWS_PER_TILE)])


def _tc1_body(cnt_ref, x_ref, w_ref, hn_ref, dis_ref):
    cnt = cnt_ref[0, :, 0:1] + cnt_ref[1, :, 0:1]
    deg = cnt + 1.0
    rid = lax.broadcasted_iota(jnp.int32, (N_PAD, 1), 0)
    dis = jnp.where(rid < N_NODES, lax.rsqrt(deg), 0.0)
    dis_ref[...] = dis
    t = jnp.dot(x_ref[...], w_ref[...], preferred_element_type=jnp.float32)
    hn = t * dis
    hn_ref[0] = hn
    hn_ref[1] = hn


def _tc_mid_body(s_ref, hn_ref, dis_ref, b_ref, w_ref, hn2_ref):
    dis = dis_ref[...]
    y = (s_ref[0] + s_ref[1] + hn_ref[0]) * dis + b_ref[...]
    h = jnp.maximum(y, 0.0)
    t = jnp.dot(h, w_ref[...], preferred_element_type=jnp.float32)
    hn2 = t * dis
    hn2_ref[0] = hn2
    hn2_ref[1] = hn2


def _tc_final_body(s_ref, hn_ref, dis_ref, b_ref, batch_ref,
                   wl1t_ref, bl1_ref, wl2t_ref, bl2_ref, out_ref):
    dis = dis_ref[...]
    y = (s_ref[0] + s_ref[1] + hn_ref[0]) * dis + b_ref[...]
    h = jnp.maximum(y, 0.0)
    gid = lax.broadcasted_iota(jnp.int32, (N_PAD, N_GRAPHS), 1)
    m = (batch_ref[...] == gid).astype(jnp.float32)
    gt = lax.dot_general(h, m, (((0,), (0,)), ((), ())),
                         preferred_element_type=jnp.float32)
    counts = jnp.sum(m, axis=0, keepdims=True)
    gt = gt / jnp.maximum(counts, 1.0)
    zt = jnp.maximum(
        jnp.dot(wl1t_ref[...], gt, preferred_element_type=jnp.float32)
        + bl1_ref[...], 0.0)
    out_ref[...] = (
        jnp.dot(wl2t_ref[...], zt, preferred_element_type=jnp.float32)
        + bl2_ref[...])


_tc1 = pl.pallas_call(
    _tc1_body,
    out_shape=(jax.ShapeDtypeStruct((2, N_PAD, D), jnp.float32),
               jax.ShapeDtypeStruct((N_PAD, 1), jnp.float32)),
)

_tc_mid = pl.pallas_call(
    _tc_mid_body,
    out_shape=jax.ShapeDtypeStruct((2, N_PAD, D), jnp.float32),
)

_tc_final = pl.pallas_call(
    _tc_final_body,
    out_shape=jax.ShapeDtypeStruct((D, N_GRAPHS), jnp.float32),
)


def kernel(x, edge_index, batch, W1, b1, W2, b2, W3, b3, Wl1, bl1, Wl2, bl2):
    f32 = jnp.float32
    src = edge_index[0].astype(jnp.int32)
    dst = edge_index[1].astype(jnp.int32)
    pad = jnp.full((E_PAD - N_EDGES,), N_NODES, dtype=jnp.int32)
    src_p = jnp.concatenate([src, pad])
    dst_p = jnp.concatenate([dst, pad])
    pad_deg = jnp.full((E_DEG_PAD - N_EDGES,), N_NODES, dtype=jnp.int32)
    dst_deg = jnp.concatenate([dst, pad_deg]).reshape(NW, DEG_NCHUNK, CHUNK)

    x_p = jnp.zeros((N_PAD, D), f32).at[:N_NODES].set(x.astype(f32))
    batch_col = jnp.full((N_PAD, 1), N_GRAPHS, jnp.int32) \
        .at[:N_NODES, 0].set(batch.astype(jnp.int32))

    zeros128 = jnp.zeros((N_PAD, D), f32)
    ones128 = jnp.ones((CHUNK, D), f32)

    b1r = b1.reshape(1, D)
    b2r = b2.reshape(1, D)
    b3r = b3.reshape(1, D)
    wl1t = Wl1.T
    bl1c = bl1.reshape(N_GRAPHS, 1)
    wl2t = jnp.zeros((D, N_GRAPHS), f32).at[:Wl2.shape[1]].set(Wl2.T)
    bl2c = jnp.zeros((D, 1), f32).at[:Wl2.shape[1], 0].set(bl2)

    cnt = _sc_degree(dst_deg, ones128, zeros128)
    hn1, dis = _tc1(cnt, x_p, W1)
    s1 = _sc_gather_scatter(hn1, src_p, dst_p, zeros128)
    hn2 = _tc_mid(s1, hn1, dis, b1r, W2)
    s2 = _sc_gather_scatter(hn2, src_p, dst_p, zeros128)
    hn3 = _tc_mid(s2, hn2, dis, b2r, W3)
    s3 = _sc_gather_scatter(hn3, src_p, dst_p, zeros128)
    out_t = _tc_final(s3, hn3, dis, b3r, batch_col, wl1t, bl1c, wl2t, bl2c)
    return out_t[:Wl2.shape[1], :].T
